# Initial kernel scaffold; baseline (speedup 1.0000x reference)
import jax
import jax.numpy as jnp
from jax.experimental import pallas as pl


def kernel(x, edge_index, W1, a_src1, a_dst1, b1, W2, a_src2, a_dst2, b2):
    raise NotImplementedError("write your pallas kernel here")


# speedup vs baseline: 30.9111x; 30.9111x over previous
import functools

import jax
import jax.numpy as jnp
from jax import lax
from jax.experimental import pallas as pl
from jax.experimental.pallas import tpu as pltpu
from jax.experimental.pallas import tpu_sc as plsc

N = 10000
D = 128
NP = 10240
BR = NP // 16
E0 = 320000
ET = E0 + N
EW = 10368
EP = 32 * EW
C = 128
NCH = EW // C
NEG = -1e30
F32 = jnp.float32


def _tc1_body(x_ref, w1_ref, as_ref, ad_ref, bd_ref, h_ref, asv_ref, adv_ref):
    i = pl.program_id(0)
    h = jnp.dot(x_ref[...], w1_ref[...], preferred_element_type=F32)
    h_ref[...] = h
    asv_ref[...] = jnp.dot(h, as_ref[...], preferred_element_type=F32)
    a_d = jnp.dot(h, ad_ref[...], preferred_element_type=F32) + bd_ref[...]
    rows = i * BR + lax.broadcasted_iota(jnp.int32, (BR, 16), 0)
    adv_ref[...] = jnp.where(rows < N, a_d, NEG)


def _tc1(xp, W1, As1, Ad1, bd1):
    return pl.pallas_call(
        _tc1_body,
        grid=(NP // BR,),
        in_specs=[
            pl.BlockSpec((BR, D), lambda i: (i, 0)),
            pl.BlockSpec((D, D), lambda i: (0, 0)),
            pl.BlockSpec((D, 16), lambda i: (0, 0)),
            pl.BlockSpec((D, 16), lambda i: (0, 0)),
            pl.BlockSpec((1, 16), lambda i: (0, 0)),
        ],
        out_specs=[
            pl.BlockSpec((BR, D), lambda i: (i, 0)),
            pl.BlockSpec((BR, 16), lambda i: (i, 0)),
            pl.BlockSpec((BR, 16), lambda i: (i, 0)),
        ],
        out_shape=[
            jax.ShapeDtypeStruct((NP, D), F32),
            jax.ShapeDtypeStruct((NP, 16), F32),
            jax.ShapeDtypeStruct((NP, 16), F32),
        ],
    )(xp, W1, As1, Ad1, bd1)


def _tc2_body(acc_ref, den_ref, p_ref, b1_ref, w2_ref, as2_ref, ad2_ref,
              bd2_ref, h2_ref, asv_ref, adv_ref):
    i = pl.program_id(0)
    accs = acc_ref[0] + acc_ref[1]
    dens = den_ref[0] + den_ref[1]
    denb = jnp.dot(dens, p_ref[...], preferred_element_type=F32)
    o = accs / (denb + 1e-16) + b1_ref[...]
    z = jnp.where(o > 0, o, jnp.exp(jnp.minimum(o, 0.0)) - 1.0)
    h2 = jnp.dot(z, w2_ref[...], preferred_element_type=F32)
    h2_ref[...] = h2
    asv_ref[...] = jnp.dot(h2, as2_ref[...], preferred_element_type=F32)
    a_d = jnp.dot(h2, ad2_ref[...], preferred_element_type=F32) + bd2_ref[...]
    rows = i * BR + lax.broadcasted_iota(jnp.int32, (BR, 16), 0)
    adv_ref[...] = jnp.where(rows < N, a_d, NEG)


def _tc2(accp, denp, P1, b1r, W2, As2, Ad2, bd2):
    return pl.pallas_call(
        _tc2_body,
        grid=(NP // BR,),
        in_specs=[
            pl.BlockSpec((2, BR, D), lambda i: (0, i, 0)),
            pl.BlockSpec((2, BR, 16), lambda i: (0, i, 0)),
            pl.BlockSpec((16, D), lambda i: (0, 0)),
            pl.BlockSpec((1, D), lambda i: (0, 0)),
            pl.BlockSpec((D, D), lambda i: (0, 0)),
            pl.BlockSpec((D, 16), lambda i: (0, 0)),
            pl.BlockSpec((D, 16), lambda i: (0, 0)),
            pl.BlockSpec((1, 16), lambda i: (0, 0)),
        ],
        out_specs=[
            pl.BlockSpec((BR, D), lambda i: (i, 0)),
            pl.BlockSpec((BR, 16), lambda i: (i, 0)),
            pl.BlockSpec((BR, 16), lambda i: (i, 0)),
        ],
        out_shape=[
            jax.ShapeDtypeStruct((NP, D), F32),
            jax.ShapeDtypeStruct((NP, 16), F32),
            jax.ShapeDtypeStruct((NP, 16), F32),
        ],
    )(accp, denp, P1, b1r, W2, As2, Ad2, bd2)


def _tc3_body(acc_ref, den_ref, p_ref, b2_ref, out_ref):
    accs = acc_ref[0] + acc_ref[1]
    dens = den_ref[0] + den_ref[1]
    denb = jnp.dot(dens, p_ref[...], preferred_element_type=F32)
    out_ref[...] = accs / (denb + 1e-16) + b2_ref[...]


def _tc3(accp, denp, P2, b2r):
    return pl.pallas_call(
        _tc3_body,
        grid=(NP // BR,),
        in_specs=[
            pl.BlockSpec((2, BR, D), lambda i: (0, i, 0)),
            pl.BlockSpec((2, BR, 16), lambda i: (0, i, 0)),
            pl.BlockSpec((16, D), lambda i: (0, 0)),
            pl.BlockSpec((1, D), lambda i: (0, 0)),
        ],
        out_specs=pl.BlockSpec((BR, D), lambda i: (i, 0)),
        out_shape=jax.ShapeDtypeStruct((NP, D), F32),
    )(accp, denp, P2, b2r)


def _make_sc(l2):
    mesh = plsc.VectorSubcoreMesh(core_axis_name="c", subcore_axis_name="s")

    def body(src_hbm, dst_hbm, h_hbm, as_hbm, ad_hbm, acc_out, den_out,
             sidx, didx, asv, adv, hv, zv, sem):
        cid = lax.axis_index("c")
        sid = lax.axis_index("s")
        wid = sid * 2 + cid
        base = sid * BR

        def zrow(i, _):
            for q in range(8):
                hv[i, pl.ds(q * 16, 16)] = jnp.zeros((16,), F32)
            zv[i, :] = jnp.zeros((16,), F32)
            return 0
        lax.fori_loop(0, C, zrow, 0)

        def zacc(i, _):
            pltpu.sync_copy(hv, acc_sh.at[pl.ds(base + i * C, C)])
            pltpu.sync_copy(zv, den_sh.at[pl.ds(base + i * C, C)])
            return 0
        lax.fori_loop(0, BR // C, zacc, 0)
        plsc.subcore_barrier()

        ebase = wid * EW

        def chunk(j, _):
            off = ebase + j * C
            pltpu.sync_copy(src_hbm.at[pl.ds(off, C)], sidx)
            pltpu.sync_copy(dst_hbm.at[pl.ds(off, C)], didx)
            cp = pltpu.async_copy(h_hbm.at[sidx], hv, sem)
            pltpu.sync_copy(as_hbm.at[sidx], asv)
            pltpu.sync_copy(ad_hbm.at[didx], adv)

            def wcalc(i, _):
                e = asv[i, :] + adv[i, :]
                e = jnp.where(e >= 0, e, 0.2 * e)
                asv[i, :] = jnp.exp(e)
                return 0
            lax.fori_loop(0, C, wcalc, 0)
            cp.wait()

            def scale(i, _):
                wrow = asv[i, :]
                if l2:
                    ws = [wrow[0]] * 8
                else:
                    ws = [wrow[q // 2] for q in range(8)]
                for q in range(8):
                    hv[i, pl.ds(q * 16, 16)] = hv[i, pl.ds(q * 16, 16)] * ws[q]
                return 0
            lax.fori_loop(0, C, scale, 0)

            pltpu.sync_copy(hv, acc_sh.at[didx], add=True)
            pltpu.sync_copy(asv, den_sh.at[didx], add=True)
            return 0
        lax.fori_loop(0, NCH, chunk, 0)

        plsc.subcore_barrier()
        pltpu.sync_copy(acc_sh.at[pl.ds(base, BR)],
                        acc_out.at[cid, pl.ds(base, BR)])
        pltpu.sync_copy(den_sh.at[pl.ds(base, BR)],
                        den_out.at[cid, pl.ds(base, BR)])

    acc_sh = None
    den_sh = None

    def body_wrap(src_hbm, dst_hbm, h_hbm, as_hbm, ad_hbm, acc_out, den_out,
                  sidx, didx, asv, adv, hv, zv,
                  acc_sh_, den_sh_, sem):
        nonlocal acc_sh, den_sh
        acc_sh = acc_sh_
        den_sh = den_sh_
        body(src_hbm, dst_hbm, h_hbm, as_hbm, ad_hbm, acc_out, den_out,
             sidx, didx, asv, adv, hv, zv, sem)

    return pl.kernel(
        body_wrap,
        out_type=[
            jax.ShapeDtypeStruct((2, NP, D), F32),
            jax.ShapeDtypeStruct((2, NP, 16), F32),
        ],
        mesh=mesh,
        compiler_params=pltpu.CompilerParams(use_tc_tiling_on_sc=False),
        scratch_types=[
            pltpu.VMEM((C,), jnp.int32),
            pltpu.VMEM((C,), jnp.int32),
            pltpu.VMEM((C, 16), F32),
            pltpu.VMEM((C, 16), F32),
            pltpu.VMEM((C, D), F32),
            pltpu.VMEM((C, 16), F32),
            pltpu.VMEM_SHARED((NP, D), F32),
            pltpu.VMEM_SHARED((NP, 16), F32),
            pltpu.SemaphoreType.DMA,
        ],
    )


_sc1 = _make_sc(l2=False)
_sc2 = _make_sc(l2=True)


def kernel(x, edge_index, W1, a_src1, a_dst1, b1, W2, a_src2, a_dst2, b2):
    loop = jnp.arange(N, dtype=jnp.int32)
    src = jnp.concatenate([edge_index[0], loop,
                           jnp.zeros((EP - ET,), jnp.int32)])
    dst = jnp.concatenate([edge_index[1], loop,
                           jnp.full((EP - ET,), N, jnp.int32)])
    xp = jnp.pad(x, ((0, NP - N), (0, 0)))
    idx = jnp.arange(D)
    As1 = jnp.zeros((D, 16), F32).at[idx, idx // 32].set(a_src1.reshape(-1))
    Ad1 = jnp.zeros((D, 16), F32).at[idx, idx // 32].set(a_dst1.reshape(-1))
    bd1 = jnp.where(jnp.arange(16) < 4, 0.0, NEG).reshape(1, 16).astype(F32)
    As2 = jnp.zeros((D, 16), F32).at[:, 0].set(a_src2[0])
    Ad2 = jnp.zeros((D, 16), F32).at[:, 0].set(a_dst2[0])
    bd2 = jnp.where(jnp.arange(16) < 1, 0.0, NEG).reshape(1, 16).astype(F32)
    P1 = jnp.zeros((16, D), F32).at[idx // 32, idx].set(1.0)
    P2 = jnp.zeros((16, D), F32).at[0, :].set(1.0)
    b1r = b1.reshape(1, D)
    b2r = b2.reshape(1, D)

    h1, as1, ad1 = _tc1(xp, W1, As1, Ad1, bd1)
    accp, denp = _sc1(src, dst, h1, as1, ad1)
    h2, as2, ad2 = _tc2(accp, denp, P1, b1r, W2, As2, Ad2, bd2)
    accp2, denp2 = _sc2(src, dst, h2, as2, ad2)
    out = _tc3(accp2, denp2, P2, b2r)
    return out[:N]


def _sc_jnp(s, d, h, as_, ad_, l2):
    e = as_[s] + ad_[d]
    e = jnp.where(e >= 0, e, 0.2 * e)
    w = jnp.exp(e)
    nh = 1 if l2 else 4
    wf = jnp.repeat(w[:, :nh], D // nh, axis=1)
    acc = jax.ops.segment_sum(wf * h[s], d, num_segments=NP)
    den = jax.ops.segment_sum(w, d, num_segments=NP)
    z = jnp.zeros_like(acc)
    zd = jnp.zeros_like(den)
    return jnp.stack([acc, z]), jnp.stack([den, zd])


# speedup vs baseline: 31.8827x; 1.0314x over previous
import jax
import jax.numpy as jnp
from jax import lax
from jax.experimental import pallas as pl
from jax.experimental.pallas import tpu as pltpu
from jax.experimental.pallas import tpu_sc as plsc

N = 10000
D = 128
NP = 10112
BR = NP // 16
E0 = 320000
ET = E0 + N
EW = 10368
EP = 32 * EW
C = 128
NCH = EW // C
NEG = -1e30
F32 = jnp.float32


def _tc1_body(x_ref, w1_ref, as_ref, ad_ref, bd_ref, h_ref, asv_ref, adv_ref):
    i = pl.program_id(0)
    h = jnp.dot(x_ref[...], w1_ref[...], preferred_element_type=F32)
    h_ref[...] = h
    asv_ref[...] = jnp.dot(h, as_ref[...], preferred_element_type=F32)
    a_d = jnp.dot(h, ad_ref[...], preferred_element_type=F32) + bd_ref[...]
    rows = i * BR + lax.broadcasted_iota(jnp.int32, (BR, 16), 0)
    adv_ref[...] = jnp.where(rows < N, a_d, NEG)


def _tc1(xp, W1, As1, Ad1, bd1):
    return pl.pallas_call(
        _tc1_body,
        grid=(NP // BR,),
        in_specs=[
            pl.BlockSpec((BR, D), lambda i: (i, 0)),
            pl.BlockSpec((D, D), lambda i: (0, 0)),
            pl.BlockSpec((D, 16), lambda i: (0, 0)),
            pl.BlockSpec((D, 16), lambda i: (0, 0)),
            pl.BlockSpec((1, 16), lambda i: (0, 0)),
        ],
        out_specs=[
            pl.BlockSpec((BR, D), lambda i: (i, 0)),
            pl.BlockSpec((BR, 16), lambda i: (i, 0)),
            pl.BlockSpec((BR, 16), lambda i: (i, 0)),
        ],
        out_shape=[
            jax.ShapeDtypeStruct((NP, D), F32),
            jax.ShapeDtypeStruct((NP, 16), F32),
            jax.ShapeDtypeStruct((NP, 16), F32),
        ],
    )(xp, W1, As1, Ad1, bd1)


def _tc2_body(acc_ref, den_ref, p_ref, b1_ref, w2_ref, as2_ref, ad2_ref,
              bd2_ref, h2_ref, asv_ref, adv_ref):
    i = pl.program_id(0)
    accs = acc_ref[0] + acc_ref[1]
    dens = den_ref[0] + den_ref[1]
    denb = jnp.dot(dens, p_ref[...], preferred_element_type=F32)
    o = accs / (denb + 1e-16) + b1_ref[...]
    z = jnp.where(o > 0, o, jnp.exp(jnp.minimum(o, 0.0)) - 1.0)
    h2 = jnp.dot(z, w2_ref[...], preferred_element_type=F32)
    h2_ref[...] = h2
    asv_ref[...] = jnp.dot(h2, as2_ref[...], preferred_element_type=F32)
    a_d = jnp.dot(h2, ad2_ref[...], preferred_element_type=F32) + bd2_ref[...]
    rows = i * BR + lax.broadcasted_iota(jnp.int32, (BR, 16), 0)
    adv_ref[...] = jnp.where(rows < N, a_d, NEG)


def _tc2(accp, denp, P1, b1r, W2, As2, Ad2, bd2):
    return pl.pallas_call(
        _tc2_body,
        grid=(NP // BR,),
        in_specs=[
            pl.BlockSpec((2, BR, D), lambda i: (0, i, 0)),
            pl.BlockSpec((2, BR, 16), lambda i: (0, i, 0)),
            pl.BlockSpec((16, D), lambda i: (0, 0)),
            pl.BlockSpec((1, D), lambda i: (0, 0)),
            pl.BlockSpec((D, D), lambda i: (0, 0)),
            pl.BlockSpec((D, 16), lambda i: (0, 0)),
            pl.BlockSpec((D, 16), lambda i: (0, 0)),
            pl.BlockSpec((1, 16), lambda i: (0, 0)),
        ],
        out_specs=[
            pl.BlockSpec((BR, D), lambda i: (i, 0)),
            pl.BlockSpec((BR, 16), lambda i: (i, 0)),
            pl.BlockSpec((BR, 16), lambda i: (i, 0)),
        ],
        out_shape=[
            jax.ShapeDtypeStruct((NP, D), F32),
            jax.ShapeDtypeStruct((NP, 16), F32),
            jax.ShapeDtypeStruct((NP, 16), F32),
        ],
    )(accp, denp, P1, b1r, W2, As2, Ad2, bd2)


def _tc3_body(acc_ref, den_ref, p_ref, b2_ref, out_ref):
    accs = acc_ref[0] + acc_ref[1]
    dens = den_ref[0] + den_ref[1]
    denb = jnp.dot(dens, p_ref[...], preferred_element_type=F32)
    out_ref[...] = accs / (denb + 1e-16) + b2_ref[...]


def _tc3(accp, denp, P2, b2r):
    return pl.pallas_call(
        _tc3_body,
        grid=(NP // BR,),
        in_specs=[
            pl.BlockSpec((2, BR, D), lambda i: (0, i, 0)),
            pl.BlockSpec((2, BR, 16), lambda i: (0, i, 0)),
            pl.BlockSpec((16, D), lambda i: (0, 0)),
            pl.BlockSpec((1, D), lambda i: (0, 0)),
        ],
        out_specs=pl.BlockSpec((BR, D), lambda i: (i, 0)),
        out_shape=jax.ShapeDtypeStruct((NP, D), F32),
    )(accp, denp, P2, b2r)


def _make_sc(l2):
    mesh = plsc.VectorSubcoreMesh(core_axis_name="c", subcore_axis_name="s")

    def body(src_hbm, dst_hbm, h_hbm, as_hbm, ad_hbm, acc_out, den_out,
             sidx, didx, asv, adv, hv, zv, semh):
        cid = lax.axis_index("c")
        sid = lax.axis_index("s")
        wid = sid * 2 + cid
        base = sid * BR
        ebase = wid * EW

        def zrow(i, _):
            for q in range(8):
                hv[0, i, pl.ds(q * 16, 16)] = jnp.zeros((16,), F32)
            zv[i, :] = jnp.zeros((16,), F32)
            return 0
        lax.fori_loop(0, C, zrow, 0)

        def zacc(i, _):
            pltpu.sync_copy(hv.at[0], acc_sh.at[pl.ds(base + i * C, C)])
            pltpu.sync_copy(zv, den_sh.at[pl.ds(base + i * C, C)])
            return 0
        lax.fori_loop(0, BR // C, zacc, 0)
        tail = BR - (BR // C) * C
        if tail:
            pltpu.sync_copy(hv.at[0, pl.ds(0, tail)],
                            acc_sh.at[pl.ds(base + (BR // C) * C, tail)])
            pltpu.sync_copy(zv.at[pl.ds(0, tail)],
                            den_sh.at[pl.ds(base + (BR // C) * C, tail)])
        plsc.subcore_barrier()

        def step(j, _):
            b = lax.rem(j, 2)
            b2 = 1 - b

            @pl.when(j < NCH)
            def _():
                off = ebase + j * C
                pltpu.sync_copy(src_hbm.at[pl.ds(off, C)], sidx.at[b])
                pltpu.sync_copy(dst_hbm.at[pl.ds(off, C)], didx.at[b])
                pltpu.async_copy(h_hbm.at[sidx.at[b]], hv.at[b], semh.at[b])

            @pl.when(j >= 1)
            def _():
                pltpu.sync_copy(as_hbm.at[sidx.at[b2]], asv)
                pltpu.sync_copy(ad_hbm.at[didx.at[b2]], adv)

                def wcalc(i, _):
                    e = asv[i, :] + adv[i, :]
                    e = jnp.where(e >= 0, e, 0.2 * e)
                    asv[i, :] = jnp.exp(e)
                    return 0
                lax.fori_loop(0, C, wcalc, 0)
                pltpu.make_async_copy(h_hbm.at[sidx.at[b2]], hv.at[b2],
                                      semh.at[b2]).wait()

                def scale(i, _):
                    wrow = asv[i, :]
                    if l2:
                        ws = [wrow[0]] * 8
                    else:
                        ws = [wrow[q // 2] for q in range(8)]
                    for q in range(8):
                        hv[b2, i, pl.ds(q * 16, 16)] = (
                            hv[b2, i, pl.ds(q * 16, 16)] * ws[q])
                    return 0
                lax.fori_loop(0, C, scale, 0)

                pltpu.sync_copy(hv.at[b2], acc_sh.at[didx.at[b2]], add=True)
                pltpu.sync_copy(asv, den_sh.at[didx.at[b2]], add=True)
            return 0
        lax.fori_loop(0, NCH + 1, step, 0)

        plsc.subcore_barrier()
        pltpu.sync_copy(acc_sh.at[pl.ds(base, BR)],
                        acc_out.at[cid, pl.ds(base, BR)])
        pltpu.sync_copy(den_sh.at[pl.ds(base, BR)],
                        den_out.at[cid, pl.ds(base, BR)])

    acc_sh = None
    den_sh = None

    def body_wrap(src_hbm, dst_hbm, h_hbm, as_hbm, ad_hbm, acc_out, den_out,
                  sidx, didx, asv, adv, hv, zv, acc_sh_, den_sh_, semh):
        nonlocal acc_sh, den_sh
        acc_sh = acc_sh_
        den_sh = den_sh_
        body(src_hbm, dst_hbm, h_hbm, as_hbm, ad_hbm, acc_out, den_out,
             sidx, didx, asv, adv, hv, zv, semh)

    return pl.kernel(
        body_wrap,
        out_type=[
            jax.ShapeDtypeStruct((2, NP, D), F32),
            jax.ShapeDtypeStruct((2, NP, 16), F32),
        ],
        mesh=mesh,
        compiler_params=pltpu.CompilerParams(use_tc_tiling_on_sc=False),
        scratch_types=[
            pltpu.VMEM((2, C), jnp.int32),
            pltpu.VMEM((2, C), jnp.int32),
            pltpu.VMEM((C, 16), F32),
            pltpu.VMEM((C, 16), F32),
            pltpu.VMEM((2, C, D), F32),
            pltpu.VMEM((C, 16), F32),
            pltpu.VMEM_SHARED((NP, D), F32),
            pltpu.VMEM_SHARED((NP, 16), F32),
            pltpu.SemaphoreType.DMA((2,)),
        ],
    )


_sc1 = _make_sc(l2=False)
_sc2 = _make_sc(l2=True)


def kernel(x, edge_index, W1, a_src1, a_dst1, b1, W2, a_src2, a_dst2, b2):
    loop = jnp.arange(N, dtype=jnp.int32)
    src = jnp.concatenate([edge_index[0], loop,
                           jnp.zeros((EP - ET,), jnp.int32)])
    dst = jnp.concatenate([edge_index[1], loop,
                           jnp.full((EP - ET,), N, jnp.int32)])
    xp = jnp.pad(x, ((0, NP - N), (0, 0)))
    idx = jnp.arange(D)
    As1 = jnp.zeros((D, 16), F32).at[idx, idx // 32].set(a_src1.reshape(-1))
    Ad1 = jnp.zeros((D, 16), F32).at[idx, idx // 32].set(a_dst1.reshape(-1))
    bd1 = jnp.where(jnp.arange(16) < 4, 0.0, NEG).reshape(1, 16).astype(F32)
    As2 = jnp.zeros((D, 16), F32).at[:, 0].set(a_src2[0])
    Ad2 = jnp.zeros((D, 16), F32).at[:, 0].set(a_dst2[0])
    bd2 = jnp.where(jnp.arange(16) < 1, 0.0, NEG).reshape(1, 16).astype(F32)
    P1 = jnp.zeros((16, D), F32).at[idx // 32, idx].set(1.0)
    P2 = jnp.zeros((16, D), F32).at[0, :].set(1.0)
    b1r = b1.reshape(1, D)
    b2r = b2.reshape(1, D)

    h1, as1, ad1 = _tc1(xp, W1, As1, Ad1, bd1)
    accp, denp = _sc1(src, dst, h1, as1, ad1)
    h2, as2, ad2 = _tc2(accp, denp, P1, b1r, W2, As2, Ad2, bd2)
    accp2, denp2 = _sc2(src, dst, h2, as2, ad2)
    out = _tc3(accp2, denp2, P2, b2r)
    return out[:N]


# speedup vs baseline: 38.0555x; 1.1936x over previous
import jax
import jax.numpy as jnp
from jax import lax
from jax.experimental import pallas as pl
from jax.experimental.pallas import tpu as pltpu
from jax.experimental.pallas import tpu_sc as plsc

N = 10000
D = 128
NP = 10112
BR = NP // 16
E0 = 320000
ET = E0 + N
EW = 10416
EP = 32 * EW
C = 112
NCH = EW // C
NEG = -1e30
F32 = jnp.float32


def _tc1_body(x_ref, w1_ref, as_ref, ad_ref, bd_ref, h_ref, asv_ref, adv_ref):
    i = pl.program_id(0)
    h = jnp.dot(x_ref[...], w1_ref[...], preferred_element_type=F32)
    h_ref[...] = h
    asv_ref[...] = jnp.dot(h, as_ref[...], preferred_element_type=F32)
    a_d = jnp.dot(h, ad_ref[...], preferred_element_type=F32) + bd_ref[...]
    rows = i * BR + lax.broadcasted_iota(jnp.int32, (BR, 16), 0)
    adv_ref[...] = jnp.where(rows < N, a_d, NEG)


def _tc1(xp, W1, As1, Ad1, bd1):
    return pl.pallas_call(
        _tc1_body,
        grid=(NP // BR,),
        in_specs=[
            pl.BlockSpec((BR, D), lambda i: (i, 0)),
            pl.BlockSpec((D, D), lambda i: (0, 0)),
            pl.BlockSpec((D, 16), lambda i: (0, 0)),
            pl.BlockSpec((D, 16), lambda i: (0, 0)),
            pl.BlockSpec((1, 16), lambda i: (0, 0)),
        ],
        out_specs=[
            pl.BlockSpec((BR, D), lambda i: (i, 0)),
            pl.BlockSpec((BR, 16), lambda i: (i, 0)),
            pl.BlockSpec((BR, 16), lambda i: (i, 0)),
        ],
        out_shape=[
            jax.ShapeDtypeStruct((NP, D), F32),
            jax.ShapeDtypeStruct((NP, 16), F32),
            jax.ShapeDtypeStruct((NP, 16), F32),
        ],
    )(xp, W1, As1, Ad1, bd1)


def _tc2_body(acc_ref, den_ref, p_ref, b1_ref, w2_ref, as2_ref, ad2_ref,
              bd2_ref, h2_ref, asv_ref, adv_ref):
    i = pl.program_id(0)
    accs = acc_ref[0] + acc_ref[1]
    dens = den_ref[0] + den_ref[1]
    denb = jnp.dot(dens, p_ref[...], preferred_element_type=F32)
    o = accs / (denb + 1e-16) + b1_ref[...]
    z = jnp.where(o > 0, o, jnp.exp(jnp.minimum(o, 0.0)) - 1.0)
    h2 = jnp.dot(z, w2_ref[...], preferred_element_type=F32)
    h2_ref[...] = h2
    asv_ref[...] = jnp.dot(h2, as2_ref[...], preferred_element_type=F32)
    a_d = jnp.dot(h2, ad2_ref[...], preferred_element_type=F32) + bd2_ref[...]
    rows = i * BR + lax.broadcasted_iota(jnp.int32, (BR, 16), 0)
    adv_ref[...] = jnp.where(rows < N, a_d, NEG)


def _tc2(accp, denp, P1, b1r, W2, As2, Ad2, bd2):
    return pl.pallas_call(
        _tc2_body,
        grid=(NP // BR,),
        in_specs=[
            pl.BlockSpec((2, BR, D), lambda i: (0, i, 0)),
            pl.BlockSpec((2, BR, 16), lambda i: (0, i, 0)),
            pl.BlockSpec((16, D), lambda i: (0, 0)),
            pl.BlockSpec((1, D), lambda i: (0, 0)),
            pl.BlockSpec((D, D), lambda i: (0, 0)),
            pl.BlockSpec((D, 16), lambda i: (0, 0)),
            pl.BlockSpec((D, 16), lambda i: (0, 0)),
            pl.BlockSpec((1, 16), lambda i: (0, 0)),
        ],
        out_specs=[
            pl.BlockSpec((BR, D), lambda i: (i, 0)),
            pl.BlockSpec((BR, 16), lambda i: (i, 0)),
            pl.BlockSpec((BR, 16), lambda i: (i, 0)),
        ],
        out_shape=[
            jax.ShapeDtypeStruct((NP, D), F32),
            jax.ShapeDtypeStruct((NP, 16), F32),
            jax.ShapeDtypeStruct((NP, 16), F32),
        ],
    )(accp, denp, P1, b1r, W2, As2, Ad2, bd2)


def _tc3_body(acc_ref, den_ref, p_ref, b2_ref, out_ref):
    accs = acc_ref[0] + acc_ref[1]
    dens = den_ref[0] + den_ref[1]
    denb = jnp.dot(dens, p_ref[...], preferred_element_type=F32)
    out_ref[...] = accs / (denb + 1e-16) + b2_ref[...]


def _tc3(accp, denp, P2, b2r):
    return pl.pallas_call(
        _tc3_body,
        grid=(NP // BR,),
        in_specs=[
            pl.BlockSpec((2, BR, D), lambda i: (0, i, 0)),
            pl.BlockSpec((2, BR, 16), lambda i: (0, i, 0)),
            pl.BlockSpec((16, D), lambda i: (0, 0)),
            pl.BlockSpec((1, D), lambda i: (0, 0)),
        ],
        out_specs=pl.BlockSpec((BR, D), lambda i: (i, 0)),
        out_shape=jax.ShapeDtypeStruct((NP, D), F32),
    )(accp, denp, P2, b2r)


def _make_sc(l2):
    mesh = plsc.VectorSubcoreMesh(core_axis_name="c", subcore_axis_name="s")

    def body(src_hbm, dst_hbm, h_hbm, as_hbm, ad_hbm, acc_out, den_out,
             sidx, didx, asv, adv, hv, zv, semh, sema, semd, semsa, semsd):
        cid = lax.axis_index("c")
        sid = lax.axis_index("s")
        wid = sid * 2 + cid
        base = sid * BR
        ebase = wid * EW

        def zrow(i, _):
            for q in range(8):
                hv[0, i, pl.ds(q * 16, 16)] = jnp.zeros((16,), F32)
            zv[i, :] = jnp.zeros((16,), F32)
            return 0
        lax.fori_loop(0, C, zrow, 0)

        def zacc(i, _):
            pltpu.sync_copy(hv.at[0], acc_sh.at[pl.ds(base + i * C, C)])
            pltpu.sync_copy(zv, den_sh.at[pl.ds(base + i * C, C)])
            return 0
        lax.fori_loop(0, BR // C, zacc, 0)
        tail = BR - (BR // C) * C
        if tail:
            pltpu.sync_copy(hv.at[0, pl.ds(0, tail)],
                            acc_sh.at[pl.ds(base + (BR // C) * C, tail)])
            pltpu.sync_copy(zv.at[pl.ds(0, tail)],
                            den_sh.at[pl.ds(base + (BR // C) * C, tail)])
        plsc.subcore_barrier()

        def step(j, _):
            b = lax.rem(j, 2)
            b2 = 1 - b

            @pl.when(j < NCH)
            def _():
                @pl.when(j >= 2)
                def _():
                    pltpu.make_async_copy(hv.at[b], acc_sh.at[didx.at[b]],
                                          semsa.at[b]).wait()
                    pltpu.make_async_copy(asv.at[b], den_sh.at[didx.at[b]],
                                          semsd.at[b]).wait()
                off = ebase + j * C
                pltpu.sync_copy(src_hbm.at[pl.ds(off, C)], sidx.at[b])
                pltpu.sync_copy(dst_hbm.at[pl.ds(off, C)], didx.at[b])
                pltpu.async_copy(h_hbm.at[sidx.at[b]], hv.at[b], semh.at[b])
                pltpu.async_copy(as_hbm.at[sidx.at[b]], asv.at[b], sema.at[b])
                pltpu.async_copy(ad_hbm.at[didx.at[b]], adv.at[b], semd.at[b])

            @pl.when(j >= 1)
            def _():
                pltpu.make_async_copy(as_hbm.at[sidx.at[b2]], asv.at[b2],
                                      sema.at[b2]).wait()
                pltpu.make_async_copy(ad_hbm.at[didx.at[b2]], adv.at[b2],
                                      semd.at[b2]).wait()

                pltpu.make_async_copy(h_hbm.at[sidx.at[b2]], hv.at[b2],
                                      semh.at[b2]).wait()

                def scale(i, _):
                    e = asv[b2, i, :] + adv[b2, i, :]
                    e = jnp.where(e >= 0, e, 0.2 * e)
                    wrow = jnp.exp(e)
                    asv[b2, i, :] = wrow
                    if l2:
                        ws = [wrow[0]] * 8
                    else:
                        ws = [wrow[q // 2] for q in range(8)]
                    for q in range(8):
                        hv[b2, i, pl.ds(q * 16, 16)] = (
                            hv[b2, i, pl.ds(q * 16, 16)] * ws[q])
                    return 0
                lax.fori_loop(0, C, scale, 0)

                pltpu.async_copy(hv.at[b2], acc_sh.at[didx.at[b2]],
                                 semsa.at[b2], add=True)
                pltpu.async_copy(asv.at[b2], den_sh.at[didx.at[b2]],
                                 semsd.at[b2], add=True)
            return 0
        lax.fori_loop(0, NCH + 1, step, 0)

        for s in range(2):
            pltpu.make_async_copy(hv.at[s], acc_sh.at[didx.at[s]],
                                  semsa.at[s]).wait()
            pltpu.make_async_copy(asv.at[s], den_sh.at[didx.at[s]],
                                  semsd.at[s]).wait()
        plsc.subcore_barrier()
        pltpu.sync_copy(acc_sh.at[pl.ds(base, BR)],
                        acc_out.at[cid, pl.ds(base, BR)])
        pltpu.sync_copy(den_sh.at[pl.ds(base, BR)],
                        den_out.at[cid, pl.ds(base, BR)])

    acc_sh = None
    den_sh = None

    def body_wrap(src_hbm, dst_hbm, h_hbm, as_hbm, ad_hbm, acc_out, den_out,
                  sidx, didx, asv, adv, hv, zv, acc_sh_, den_sh_, semh,
                  sema, semd, semsa, semsd):
        nonlocal acc_sh, den_sh
        acc_sh = acc_sh_
        den_sh = den_sh_
        body(src_hbm, dst_hbm, h_hbm, as_hbm, ad_hbm, acc_out, den_out,
             sidx, didx, asv, adv, hv, zv, semh, sema, semd, semsa, semsd)

    return pl.kernel(
        body_wrap,
        out_type=[
            jax.ShapeDtypeStruct((2, NP, D), F32),
            jax.ShapeDtypeStruct((2, NP, 16), F32),
        ],
        mesh=mesh,
        compiler_params=pltpu.CompilerParams(use_tc_tiling_on_sc=False),
        scratch_types=[
            pltpu.VMEM((2, C), jnp.int32),
            pltpu.VMEM((2, C), jnp.int32),
            pltpu.VMEM((2, C, 16), F32),
            pltpu.VMEM((2, C, 16), F32),
            pltpu.VMEM((2, C, D), F32),
            pltpu.VMEM((C, 16), F32),
            pltpu.VMEM_SHARED((NP, D), F32),
            pltpu.VMEM_SHARED((NP, 16), F32),
            pltpu.SemaphoreType.DMA((2,)),
            pltpu.SemaphoreType.DMA((2,)),
            pltpu.SemaphoreType.DMA((2,)),
            pltpu.SemaphoreType.DMA((2,)),
            pltpu.SemaphoreType.DMA((2,)),
        ],
    )


_sc1 = _make_sc(l2=False)
_sc2 = _make_sc(l2=True)


def kernel(x, edge_index, W1, a_src1, a_dst1, b1, W2, a_src2, a_dst2, b2):
    loop = jnp.arange(N, dtype=jnp.int32)
    src = jnp.concatenate([edge_index[0], loop,
                           jnp.zeros((EP - ET,), jnp.int32)])
    dst = jnp.concatenate([edge_index[1], loop,
                           jnp.full((EP - ET,), N, jnp.int32)])
    xp = jnp.pad(x, ((0, NP - N), (0, 0)))
    idx = jnp.arange(D)
    As1 = jnp.zeros((D, 16), F32).at[idx, idx // 32].set(a_src1.reshape(-1))
    Ad1 = jnp.zeros((D, 16), F32).at[idx, idx // 32].set(a_dst1.reshape(-1))
    bd1 = jnp.where(jnp.arange(16) < 4, 0.0, NEG).reshape(1, 16).astype(F32)
    As2 = jnp.zeros((D, 16), F32).at[:, 0].set(a_src2[0])
    Ad2 = jnp.zeros((D, 16), F32).at[:, 0].set(a_dst2[0])
    bd2 = jnp.where(jnp.arange(16) < 1, 0.0, NEG).reshape(1, 16).astype(F32)
    P1 = jnp.zeros((16, D), F32).at[idx // 32, idx].set(1.0)
    P2 = jnp.zeros((16, D), F32).at[0, :].set(1.0)
    b1r = b1.reshape(1, D)
    b2r = b2.reshape(1, D)

    h1, as1, ad1 = _tc1(xp, W1, As1, Ad1, bd1)
    accp, denp = _sc1(src, dst, h1, as1, ad1)
    h2, as2, ad2 = _tc2(accp, denp, P1, b1r, W2, As2, Ad2, bd2)
    accp2, denp2 = _sc2(src, dst, h2, as2, ad2)
    out = _tc3(accp2, denp2, P2, b2r)
    return out[:N]
